# baseline (speedup 1.0000x reference)
import functools

import jax
import jax.numpy as jnp
from jax import lax
from jax.experimental import pallas as pl
from jax.experimental.pallas import tpu as pltpu
from jax.experimental.pallas import tpu_sc as plsc

N = 10000
E = 320000
D = 128
H = 16
OUT = 40

NC, NS = 2, 16
NW = NC * NS
NPAD = 10240
CH = 80

EPW = E // NW
NCH = EPW // CH
RPS = NPAD // NS

_F32 = jnp.float32


@functools.lru_cache(maxsize=None)
def _mesh():
    return plsc.VectorSubcoreMesh(
        core_axis_name="c", subcore_axis_name="s",
        num_cores=NC, num_subcores=NS)


def _zero_f32(ref, nrows, width):
    zeros = jnp.zeros((16,), _F32)

    def body(i, _):
        for k in range(width // 16):
            ref[i, pl.ds(k * 16, 16)] = zeros
        return 0

    lax.fori_loop(0, nrows, body, 0)


def _newton_rsqrt_inplace(buf, n16):
    magic = jnp.full((16,), 0x5F3759DF, jnp.int32)

    def newt(i, _):
        d = buf[pl.ds(i * 16, 16)] + 1.0
        iy = magic - lax.shift_right_arithmetic(
            lax.bitcast_convert_type(d, jnp.int32),
            jnp.full((16,), 1, jnp.int32))
        y = lax.bitcast_convert_type(iy, _F32)
        for _ in range(3):
            y = y * (1.5 - 0.5 * d * y * y)
        buf[pl.ds(i * 16, 16)] = y
        return 0

    lax.fori_loop(0, n16, newt, 0)


NBUF = 8


def _edge_loop(table_sh, acc_sh, sbuf, dbuf, rows, gsem, ssem):
    def sidx(j):
        return sbuf.at[pl.ds(j * CH, CH)]

    def didx(j):
        return dbuf.at[pl.ds(j * CH, CH)]

    for j0 in range(NBUF - 1):
        pltpu.async_copy(table_sh.at[sidx(j0)], rows.at[j0], gsem)

    def chunk(j, _):
        b = j & (NBUF - 1)
        pltpu.make_async_copy(table_sh.at[sidx(j)], rows.at[b], gsem).wait()
        pltpu.async_copy(rows.at[b], acc_sh.at[didx(j)], ssem, add=True)

        @pl.when((j >= 1) & (j + NBUF - 1 < NCH))
        def _():
            pltpu.make_async_copy(
                rows.at[(j - 1) & (NBUF - 1)], acc_sh.at[didx(j - 1)],
                ssem).wait()

        @pl.when(j + NBUF - 1 < NCH)
        def _():
            pltpu.async_copy(
                table_sh.at[sidx(j + NBUF - 1)],
                rows.at[(j + NBUF - 1) & (NBUF - 1)], gsem)

        return 0

    lax.fori_loop(0, NCH, chunk, 0)
    for _ in range(NBUF):
        pltpu.make_async_copy(rows.at[0], acc_sh.at[didx(0)], ssem).wait()


def _sc_layer1_body(xw_hbm, ei_f, part, dinv_out,
                    y_sh, acc_sh, hist_sh, hbuf, sbuf, dbuf, rows, obuf,
                    dvbuf, ones_v, gsem, ssem):
    c = lax.axis_index("c")
    s = lax.axis_index("s")
    wid = s * NC + c
    zeros16 = jnp.zeros((16,), _F32)
    _zero_f32(obuf, RPS, H)
    pltpu.sync_copy(obuf, acc_sh.at[pl.ds(s * RPS, RPS)])

    def zh(i, _):
        dvbuf[pl.ds(i * 16, 16)] = zeros16
        return 0

    lax.fori_loop(0, RPS // 16, zh, 0)
    pltpu.sync_copy(dvbuf, hist_sh.at[pl.ds(s * RPS, RPS)])
    for k in range(CH // 16):
        ones_v[pl.ds(k * 16, 16)] = jnp.ones((16,), _F32)
    plsc.subcore_barrier()

    pltpu.sync_copy(ei_f.at[pl.ds(E + s * 2 * EPW, 2 * EPW)], hbuf)

    def hidx(j):
        return hbuf.at[pl.ds(j * CH, CH)]

    def hchunk(j, _):
        pltpu.async_copy(ones_v, hist_sh.at[hidx(j)], gsem, add=True)

        @pl.when(j >= 12)
        def _():
            pltpu.make_async_copy(ones_v, hist_sh.at[hidx(0)], gsem).wait()

        return 0

    lax.fori_loop(0, 2 * NCH, hchunk, 0)
    for _ in range(12):
        pltpu.make_async_copy(ones_v, hist_sh.at[hidx(0)], gsem).wait()
    plsc.subcore_barrier()

    pltpu.sync_copy(hist_sh.at[pl.ds(s * RPS, RPS)], dvbuf)
    _newton_rsqrt_inplace(dvbuf, RPS // 16)

    @pl.when(c == 0)
    def _():
        pltpu.sync_copy(dvbuf, dinv_out.at[pl.ds(s * RPS, RPS)])

    pltpu.sync_copy(xw_hbm.at[pl.ds(s * RPS, RPS)], obuf)

    def scale(i, _):
        d16 = plsc.load_gather(dvbuf, [jnp.full((16,), i, jnp.int32)])
        obuf[i, :] = obuf[i, :] * d16
        return 0

    lax.fori_loop(0, RPS, scale, 0)
    pltpu.sync_copy(obuf, y_sh.at[pl.ds(s * RPS, RPS)])
    plsc.subcore_barrier()

    pltpu.sync_copy(ei_f.at[pl.ds(wid * EPW, EPW)], sbuf)
    pltpu.sync_copy(ei_f.at[pl.ds(E + wid * EPW, EPW)], dbuf)
    _edge_loop(y_sh, acc_sh, sbuf, dbuf, rows, gsem, ssem)
    plsc.subcore_barrier()

    pltpu.sync_copy(acc_sh.at[pl.ds(s * RPS, RPS)], obuf)
    pltpu.sync_copy(obuf, part.at[c, pl.ds(s * RPS, RPS)])


@functools.lru_cache(maxsize=None)
def _get_layer1():
    return pl.kernel(
        _sc_layer1_body,
        out_type=(jax.ShapeDtypeStruct((NC, NPAD, H), _F32),
                  jax.ShapeDtypeStruct((NPAD,), _F32)),
        mesh=_mesh(),
        scratch_types=[
            pltpu.VMEM_SHARED((NPAD, H), _F32),
            pltpu.VMEM_SHARED((NPAD, H), _F32),
            pltpu.VMEM_SHARED((NPAD,), _F32),
            pltpu.VMEM((2 * EPW,), jnp.int32),
            pltpu.VMEM((EPW,), jnp.int32),
            pltpu.VMEM((EPW,), jnp.int32),
            pltpu.VMEM((NBUF, CH, H), _F32),
            pltpu.VMEM((RPS, H), _F32),
            pltpu.VMEM((RPS,), _F32),
            pltpu.VMEM((CH,), _F32),
            pltpu.SemaphoreType.DMA,
            pltpu.SemaphoreType.DMA,
        ],
        compiler_params=pltpu.CompilerParams(
            use_tc_tiling_on_sc=False, needs_layout_passes=False),
    )


@functools.lru_cache(maxsize=None)
def _make_scatter(K):
    def scatter(y_hbm, ei_f, part, acc_sh, sbuf, dbuf, rows,
                obuf, gsem, ssem):
        c = lax.axis_index("c")
        s = lax.axis_index("s")
        wid = s * NC + c
        _zero_f32(obuf, RPS, K)
        pltpu.sync_copy(obuf, acc_sh.at[pl.ds(s * RPS, RPS)])
        plsc.subcore_barrier()

        pltpu.sync_copy(ei_f.at[pl.ds(wid * EPW, EPW)], sbuf)
        pltpu.sync_copy(ei_f.at[pl.ds(E + wid * EPW, EPW)], dbuf)
        _edge_loop(y_hbm, acc_sh, sbuf, dbuf, rows, gsem, ssem)
        plsc.subcore_barrier()

        pltpu.sync_copy(acc_sh.at[pl.ds(s * RPS, RPS)], obuf)
        pltpu.sync_copy(obuf, part.at[c, pl.ds(s * RPS, RPS)])

    return pl.kernel(
        scatter,
        out_type=jax.ShapeDtypeStruct((NC, NPAD, K), _F32),
        mesh=_mesh(),
        scratch_types=[
            pltpu.VMEM_SHARED((NPAD, K), _F32),
            pltpu.VMEM((EPW,), jnp.int32),
            pltpu.VMEM((EPW,), jnp.int32),
            pltpu.VMEM((NBUF, CH, K), _F32),
            pltpu.VMEM((RPS, K), _F32),
            pltpu.SemaphoreType.DMA,
            pltpu.SemaphoreType.DMA,
        ],
        compiler_params=pltpu.CompilerParams(
            use_tc_tiling_on_sc=False, needs_layout_passes=False),
    )


def _tc1_body(x_ref, w_ref, y_ref):
    y_ref[...] = jnp.dot(x_ref[...], w_ref[...],
                         preferred_element_type=_F32)


def _tc2_body(p, xw, dinv, b1, w2, y2_ref):
    d = dinv[...]
    h = jnp.maximum(d * (p[0] + p[1] + d * xw[...]) + b1[...], 0.0)
    y2_ref[...] = jnp.dot(h, w2[...], preferred_element_type=_F32) * d


def _tc3_body(q, y2, dinv, b2, out_ref):
    z = dinv[...] * (q[0] + q[1] + y2[...]) + b2[...]
    m = jnp.max(z, axis=1, keepdims=True)
    e = jnp.exp(z - m)
    lse = jnp.log(jnp.sum(e, axis=1, keepdims=True)) + m
    out_ref[...] = z - lse


def kernel(x, edge_index, W1, b1, W2, b2):
    ei_f = edge_index.reshape(2 * E)
    xpad = jnp.pad(x, ((0, NPAD - N), (0, 0)))

    xw = pl.pallas_call(
        _tc1_body,
        grid=(8,),
        in_specs=[pl.BlockSpec((NPAD // 8, D), lambda i: (i, 0)),
                  pl.BlockSpec((D, H), lambda i: (0, 0))],
        out_specs=pl.BlockSpec((NPAD // 8, H), lambda i: (i, 0)),
        out_shape=jax.ShapeDtypeStruct((NPAD, H), _F32),
    )(xpad, W1)

    p, dinv = _get_layer1()(xw, ei_f)
    dinv2 = dinv.reshape(NPAD, 1)

    w2p = jnp.pad(W2, ((0, 0), (0, 48 - OUT)))
    b2p = jnp.concatenate([b2, jnp.full((48 - OUT,), -1e30, _F32)])
    RB = NPAD // 8
    y2 = pl.pallas_call(
        _tc2_body,
        grid=(8,),
        in_specs=[pl.BlockSpec((NC, RB, H), lambda i: (0, i, 0)),
                  pl.BlockSpec((RB, H), lambda i: (i, 0)),
                  pl.BlockSpec((RB, 1), lambda i: (i, 0)),
                  pl.BlockSpec((1, H), lambda i: (0, 0)),
                  pl.BlockSpec((H, 48), lambda i: (0, 0))],
        out_specs=pl.BlockSpec((RB, 48), lambda i: (i, 0)),
        out_shape=jax.ShapeDtypeStruct((NPAD, 48), _F32),
    )(p, xw, dinv2, b1.reshape(1, H), w2p)

    q = _make_scatter(48)(y2, ei_f)

    outp = pl.pallas_call(
        _tc3_body,
        grid=(8,),
        in_specs=[pl.BlockSpec((NC, RB, 48), lambda i: (0, i, 0)),
                  pl.BlockSpec((RB, 48), lambda i: (i, 0)),
                  pl.BlockSpec((RB, 1), lambda i: (i, 0)),
                  pl.BlockSpec((1, 48), lambda i: (0, 0))],
        out_specs=pl.BlockSpec((RB, 48), lambda i: (i, 0)),
        out_shape=jax.ShapeDtypeStruct((NPAD, 48), _F32),
    )(q, y2, dinv2, b2p.reshape(1, 48))

    return outp[:N, :OUT]


# speedup vs baseline: 1.0162x; 1.0162x over previous
import functools

import jax
import jax.numpy as jnp
from jax import lax
from jax.experimental import pallas as pl
from jax.experimental.pallas import tpu as pltpu
from jax.experimental.pallas import tpu_sc as plsc

N = 10000
E = 320000
D = 128
H = 16
OUT = 40

NC, NS = 2, 16
NW = NC * NS
NPAD = 10240
CH = 80

EPW = E // NW
NCH = EPW // CH
RPS = NPAD // NS

_F32 = jnp.float32


@functools.lru_cache(maxsize=None)
def _mesh():
    return plsc.VectorSubcoreMesh(
        core_axis_name="c", subcore_axis_name="s",
        num_cores=NC, num_subcores=NS)


def _zero_f32(ref, nrows, width):
    zeros = jnp.zeros((16,), _F32)

    def body(i, _):
        for k in range(width // 16):
            ref[i, pl.ds(k * 16, 16)] = zeros
        return 0

    lax.fori_loop(0, nrows, body, 0)


def _newton_rsqrt_inplace(buf, n16):
    magic = jnp.full((16,), 0x5F3759DF, jnp.int32)

    def newt(i, _):
        d = buf[pl.ds(i * 16, 16)] + 1.0
        iy = magic - lax.shift_right_arithmetic(
            lax.bitcast_convert_type(d, jnp.int32),
            jnp.full((16,), 1, jnp.int32))
        y = lax.bitcast_convert_type(iy, _F32)
        for _ in range(3):
            y = y * (1.5 - 0.5 * d * y * y)
        buf[pl.ds(i * 16, 16)] = y
        return 0

    lax.fori_loop(0, n16, newt, 0)


NBUF = 8


def _edge_loop(table_sh, acc_sh, sbuf, dbuf, rows, gsem, ssem):
    def sidx(j):
        return sbuf.at[pl.ds(j * CH, CH)]

    def didx(j):
        return dbuf.at[pl.ds(j * CH, CH)]

    for j0 in range(NBUF - 1):
        pltpu.async_copy(table_sh.at[sidx(j0)], rows.at[j0], gsem)

    def chunk(j, _):
        b = j & (NBUF - 1)
        pltpu.make_async_copy(table_sh.at[sidx(j)], rows.at[b], gsem).wait()
        pltpu.async_copy(rows.at[b], acc_sh.at[didx(j)], ssem, add=True)

        @pl.when((j >= 1) & (j + NBUF - 1 < NCH))
        def _():
            pltpu.make_async_copy(
                rows.at[(j - 1) & (NBUF - 1)], acc_sh.at[didx(j - 1)],
                ssem).wait()

        @pl.when(j + NBUF - 1 < NCH)
        def _():
            pltpu.async_copy(
                table_sh.at[sidx(j + NBUF - 1)],
                rows.at[(j + NBUF - 1) & (NBUF - 1)], gsem)

        return 0

    lax.fori_loop(0, NCH, chunk, 0)
    for _ in range(NBUF):
        pltpu.make_async_copy(rows.at[0], acc_sh.at[didx(0)], ssem).wait()


def _sc_layer1_body(xw_hbm, ei_f, part, dinv_out,
                    y_sh, acc_sh, hist_sh, hbuf, sbuf, dbuf, rows, obuf,
                    dvbuf, ones_v, gsem, ssem):
    c = lax.axis_index("c")
    s = lax.axis_index("s")
    wid = s * NC + c
    zeros16 = jnp.zeros((16,), _F32)
    _zero_f32(obuf, RPS, H)
    pltpu.sync_copy(obuf, acc_sh.at[pl.ds(s * RPS, RPS)])

    def zh(i, _):
        dvbuf[pl.ds(i * 16, 16)] = zeros16
        return 0

    lax.fori_loop(0, RPS // 16, zh, 0)
    pltpu.sync_copy(dvbuf, hist_sh.at[pl.ds(s * RPS, RPS)])
    for k in range(CH // 16):
        ones_v[pl.ds(k * 16, 16)] = jnp.ones((16,), _F32)
    plsc.subcore_barrier()

    pltpu.sync_copy(ei_f.at[pl.ds(E + s * 2 * EPW, 2 * EPW)], hbuf)

    def hidx(j):
        return hbuf.at[pl.ds(j * CH, CH)]

    def hchunk(j, _):
        pltpu.async_copy(ones_v, hist_sh.at[hidx(j)], gsem, add=True)

        @pl.when(j >= 12)
        def _():
            pltpu.make_async_copy(ones_v, hist_sh.at[hidx(0)], gsem).wait()

        return 0

    lax.fori_loop(0, 2 * NCH, hchunk, 0)
    for _ in range(12):
        pltpu.make_async_copy(ones_v, hist_sh.at[hidx(0)], gsem).wait()
    plsc.subcore_barrier()

    pltpu.sync_copy(hist_sh.at[pl.ds(s * RPS, RPS)], dvbuf)
    _newton_rsqrt_inplace(dvbuf, RPS // 16)

    @pl.when(c == 0)
    def _():
        pltpu.sync_copy(dvbuf, dinv_out.at[pl.ds(s * RPS, RPS)])

    pltpu.sync_copy(xw_hbm.at[pl.ds(s * RPS, RPS)], obuf)

    def scale(i, _):
        d16 = plsc.load_gather(dvbuf, [jnp.full((16,), i, jnp.int32)])
        obuf[i, :] = obuf[i, :] * d16
        return 0

    lax.fori_loop(0, RPS, scale, 0)
    pltpu.sync_copy(obuf, y_sh.at[pl.ds(s * RPS, RPS)])
    plsc.subcore_barrier()

    pltpu.sync_copy(ei_f.at[pl.ds(wid * EPW, EPW)], sbuf)
    pltpu.sync_copy(ei_f.at[pl.ds(E + wid * EPW, EPW)], dbuf)
    _edge_loop(y_sh, acc_sh, sbuf, dbuf, rows, gsem, ssem)
    plsc.subcore_barrier()

    pltpu.sync_copy(acc_sh.at[pl.ds(s * RPS, RPS)], obuf)
    pltpu.sync_copy(obuf, part.at[c, pl.ds(s * RPS, RPS)])


@functools.lru_cache(maxsize=None)
def _get_layer1():
    return pl.kernel(
        _sc_layer1_body,
        out_type=(jax.ShapeDtypeStruct((NC, NPAD, H), _F32),
                  jax.ShapeDtypeStruct((NPAD,), _F32)),
        mesh=_mesh(),
        scratch_types=[
            pltpu.VMEM_SHARED((NPAD, H), _F32),
            pltpu.VMEM_SHARED((NPAD, H), _F32),
            pltpu.VMEM_SHARED((NPAD,), _F32),
            pltpu.VMEM((2 * EPW,), jnp.int32),
            pltpu.VMEM((EPW,), jnp.int32),
            pltpu.VMEM((EPW,), jnp.int32),
            pltpu.VMEM((NBUF, CH, H), _F32),
            pltpu.VMEM((RPS, H), _F32),
            pltpu.VMEM((RPS,), _F32),
            pltpu.VMEM((CH,), _F32),
            pltpu.SemaphoreType.DMA,
            pltpu.SemaphoreType.DMA,
        ],
        compiler_params=pltpu.CompilerParams(
            use_tc_tiling_on_sc=False, needs_layout_passes=False),
    )


@functools.lru_cache(maxsize=None)
def _make_scatter(K):
    def scatter(y_hbm, ei_f, part, acc_sh, sbuf, dbuf, rows,
                obuf, gsem, ssem):
        c = lax.axis_index("c")
        s = lax.axis_index("s")
        wid = s * NC + c
        _zero_f32(obuf, RPS, K)
        pltpu.sync_copy(obuf, acc_sh.at[pl.ds(s * RPS, RPS)])
        plsc.subcore_barrier()

        pltpu.sync_copy(ei_f.at[pl.ds(wid * EPW, EPW)], sbuf)
        pltpu.sync_copy(ei_f.at[pl.ds(E + wid * EPW, EPW)], dbuf)
        _edge_loop(y_hbm, acc_sh, sbuf, dbuf, rows, gsem, ssem)
        plsc.subcore_barrier()

        pltpu.sync_copy(acc_sh.at[pl.ds(s * RPS, RPS)], obuf)
        pltpu.sync_copy(obuf, part.at[c, pl.ds(s * RPS, RPS)])

    return pl.kernel(
        scatter,
        out_type=jax.ShapeDtypeStruct((NC, NPAD, K), _F32),
        mesh=_mesh(),
        scratch_types=[
            pltpu.VMEM_SHARED((NPAD, K), _F32),
            pltpu.VMEM((EPW,), jnp.int32),
            pltpu.VMEM((EPW,), jnp.int32),
            pltpu.VMEM((NBUF, CH, K), _F32),
            pltpu.VMEM((RPS, K), _F32),
            pltpu.SemaphoreType.DMA,
            pltpu.SemaphoreType.DMA,
        ],
        compiler_params=pltpu.CompilerParams(
            use_tc_tiling_on_sc=False, needs_layout_passes=False),
    )


def _tc1_body(x_ref, w_ref, y_ref):
    y_ref[...] = jnp.dot(x_ref[...], w_ref[...],
                         preferred_element_type=_F32)


def _tc2_body(p, xw, dinv, b1, w2, y2_ref):
    d = dinv[...]
    h = jnp.maximum(d * (p[0] + p[1] + d * xw[...]) + b1[...], 0.0)
    y2_ref[...] = jnp.dot(h, w2[...], preferred_element_type=_F32) * d


def _tc3_body(q, y2, dinv, b2, out_ref):
    z = dinv[...] * (q[0] + q[1] + y2[...]) + b2[...]
    m = jnp.max(z, axis=1, keepdims=True)
    e = jnp.exp(z - m)
    lse = jnp.log(jnp.sum(e, axis=1, keepdims=True)) + m
    out_ref[...] = z - lse


def kernel(x, edge_index, W1, b1, W2, b2):
    ei_f = edge_index.reshape(2 * E)
    xpad = jnp.pad(x, ((0, NPAD - N), (0, 0)))

    xw = pl.pallas_call(
        _tc1_body,
        out_shape=jax.ShapeDtypeStruct((NPAD, H), _F32),
    )(xpad, W1)

    p, dinv = _get_layer1()(xw, ei_f)
    dinv2 = dinv.reshape(NPAD, 1)

    w2p = jnp.pad(W2, ((0, 0), (0, 48 - OUT)))
    b2p = jnp.concatenate([b2, jnp.full((48 - OUT,), -1e30, _F32)])
    y2 = pl.pallas_call(
        _tc2_body,
        out_shape=jax.ShapeDtypeStruct((NPAD, 48), _F32),
    )(p, xw, dinv2, b1.reshape(1, H), w2p)

    q = _make_scatter(48)(y2, ei_f)

    outp = pl.pallas_call(
        _tc3_body,
        out_shape=jax.ShapeDtypeStruct((NPAD, 48), _F32),
    )(q, y2, dinv2, b2p.reshape(1, 48))

    return outp[:N, :OUT]


# speedup vs baseline: 1.0486x; 1.0319x over previous
import functools

import jax
import jax.numpy as jnp
from jax import lax
from jax.experimental import pallas as pl
from jax.experimental.pallas import tpu as pltpu
from jax.experimental.pallas import tpu_sc as plsc

N = 10000
E = 320000
D = 128
H = 16
OUT = 40

NC, NS = 2, 16
NW = NC * NS
NPAD = 10240
CH = 80

EPW = E // NW
NCH = EPW // CH
RPS = NPAD // NS

_F32 = jnp.float32


@functools.lru_cache(maxsize=None)
def _mesh():
    return plsc.VectorSubcoreMesh(
        core_axis_name="c", subcore_axis_name="s",
        num_cores=NC, num_subcores=NS)


def _zero_f32(ref, nrows, width):
    zeros = jnp.zeros((16,), _F32)

    def body(i, _):
        for k in range(width // 16):
            ref[i, pl.ds(k * 16, 16)] = zeros
        return 0

    lax.fori_loop(0, nrows, body, 0)


def _newton_rsqrt_inplace(buf, n16):
    magic = jnp.full((16,), 0x5F3759DF, jnp.int32)

    def newt(i, _):
        d = buf[pl.ds(i * 16, 16)] + 1.0
        iy = magic - lax.shift_right_arithmetic(
            lax.bitcast_convert_type(d, jnp.int32),
            jnp.full((16,), 1, jnp.int32))
        y = lax.bitcast_convert_type(iy, _F32)
        for _ in range(3):
            y = y * (1.5 - 0.5 * d * y * y)
        buf[pl.ds(i * 16, 16)] = y
        return 0

    lax.fori_loop(0, n16, newt, 0)


NBUF = 8


def _edge_loop(table_sh, acc_sh, sbuf, dbuf, rows, gsem, ssem):
    def sidx(j):
        return sbuf.at[pl.ds(j * CH, CH)]

    def didx(j):
        return dbuf.at[pl.ds(j * CH, CH)]

    for j0 in range(NBUF - 1):
        pltpu.async_copy(table_sh.at[sidx(j0)], rows.at[j0], gsem)

    def chunk(j, _):
        b = j & (NBUF - 1)
        pltpu.make_async_copy(table_sh.at[sidx(j)], rows.at[b], gsem).wait()
        pltpu.async_copy(rows.at[b], acc_sh.at[didx(j)], ssem, add=True)

        @pl.when((j >= 1) & (j + NBUF - 1 < NCH))
        def _():
            pltpu.make_async_copy(
                rows.at[(j - 1) & (NBUF - 1)], acc_sh.at[didx(j - 1)],
                ssem).wait()

        @pl.when(j + NBUF - 1 < NCH)
        def _():
            pltpu.async_copy(
                table_sh.at[sidx(j + NBUF - 1)],
                rows.at[(j + NBUF - 1) & (NBUF - 1)], gsem)

        return 0

    lax.fori_loop(0, NCH, chunk, 0)
    for _ in range(NBUF):
        pltpu.make_async_copy(rows.at[0], acc_sh.at[didx(0)], ssem).wait()


def _sc_layer1_body(xw_hbm, ei_f, part, dinv_out,
                    y_sh, acc_sh, hist_sh, hbuf, sbuf, dbuf, rows, obuf,
                    dvbuf, ones_v, gsem, ssem):
    c = lax.axis_index("c")
    s = lax.axis_index("s")
    wid = s * NC + c
    zeros16 = jnp.zeros((16,), _F32)
    _zero_f32(obuf, RPS, H)
    pltpu.sync_copy(obuf, acc_sh.at[pl.ds(s * RPS, RPS)])

    def zh(i, _):
        dvbuf[pl.ds(i * 16, 16)] = zeros16
        return 0

    lax.fori_loop(0, RPS // 16, zh, 0)
    pltpu.sync_copy(dvbuf, hist_sh.at[pl.ds(s * RPS, RPS)])
    for k in range(CH // 16):
        ones_v[pl.ds(k * 16, 16)] = jnp.ones((16,), _F32)
    plsc.subcore_barrier()

    pltpu.sync_copy(ei_f.at[pl.ds(E + s * 2 * EPW, 2 * EPW)], hbuf)

    def hidx(j):
        return hbuf.at[pl.ds(j * CH, CH)]

    def hchunk(j, _):
        pltpu.async_copy(ones_v, hist_sh.at[hidx(j)], gsem, add=True)

        @pl.when(j >= 12)
        def _():
            pltpu.make_async_copy(ones_v, hist_sh.at[hidx(0)], gsem).wait()

        return 0

    lax.fori_loop(0, 2 * NCH, hchunk, 0)
    for _ in range(12):
        pltpu.make_async_copy(ones_v, hist_sh.at[hidx(0)], gsem).wait()
    plsc.subcore_barrier()

    pltpu.sync_copy(hist_sh.at[pl.ds(s * RPS, RPS)], dvbuf)
    _newton_rsqrt_inplace(dvbuf, RPS // 16)

    @pl.when(c == 0)
    def _():
        pltpu.sync_copy(dvbuf, dinv_out.at[pl.ds(s * RPS, RPS)])

    pltpu.sync_copy(xw_hbm.at[pl.ds(s * RPS, RPS)], obuf)

    def scale(i, _):
        d16 = plsc.load_gather(dvbuf, [jnp.full((16,), i, jnp.int32)])
        obuf[i, :] = obuf[i, :] * d16
        return 0

    lax.fori_loop(0, RPS, scale, 0)
    pltpu.sync_copy(obuf, y_sh.at[pl.ds(s * RPS, RPS)])
    plsc.subcore_barrier()

    pltpu.sync_copy(ei_f.at[pl.ds(wid * EPW, EPW)], sbuf)
    pltpu.sync_copy(ei_f.at[pl.ds(E + wid * EPW, EPW)], dbuf)
    _edge_loop(y_sh, acc_sh, sbuf, dbuf, rows, gsem, ssem)
    plsc.subcore_barrier()

    pltpu.sync_copy(acc_sh.at[pl.ds(s * RPS, RPS)], obuf)
    pltpu.sync_copy(obuf, part.at[c, pl.ds(s * RPS, RPS)])


@functools.lru_cache(maxsize=None)
def _get_layer1():
    return pl.kernel(
        _sc_layer1_body,
        out_type=(jax.ShapeDtypeStruct((NC, NPAD, H), _F32),
                  jax.ShapeDtypeStruct((NPAD,), _F32)),
        mesh=_mesh(),
        scratch_types=[
            pltpu.VMEM_SHARED((NPAD, H), _F32),
            pltpu.VMEM_SHARED((NPAD, H), _F32),
            pltpu.VMEM_SHARED((NPAD,), _F32),
            pltpu.VMEM((2 * EPW,), jnp.int32),
            pltpu.VMEM((EPW,), jnp.int32),
            pltpu.VMEM((EPW,), jnp.int32),
            pltpu.VMEM((NBUF, CH, H), _F32),
            pltpu.VMEM((RPS, H), _F32),
            pltpu.VMEM((RPS,), _F32),
            pltpu.VMEM((CH,), _F32),
            pltpu.SemaphoreType.DMA,
            pltpu.SemaphoreType.DMA,
        ],
        compiler_params=pltpu.CompilerParams(
            use_tc_tiling_on_sc=False, needs_layout_passes=False),
    )


@functools.lru_cache(maxsize=None)
def _make_scatter(K):
    def scatter(y_hbm, ei_f, part, acc_sh, sbuf, dbuf, rows,
                obuf, gsem, ssem):
        c = lax.axis_index("c")
        s = lax.axis_index("s")
        wid = s * NC + c
        _zero_f32(obuf, RPS, K)
        pltpu.sync_copy(obuf, acc_sh.at[pl.ds(s * RPS, RPS)])
        plsc.subcore_barrier()

        pltpu.sync_copy(ei_f.at[pl.ds(wid * EPW, EPW)], sbuf)
        pltpu.sync_copy(ei_f.at[pl.ds(E + wid * EPW, EPW)], dbuf)
        _edge_loop(y_hbm, acc_sh, sbuf, dbuf, rows, gsem, ssem)
        plsc.subcore_barrier()

        pltpu.sync_copy(acc_sh.at[pl.ds(s * RPS, RPS)], obuf)
        pltpu.sync_copy(obuf, part.at[c, pl.ds(s * RPS, RPS)])

    return pl.kernel(
        scatter,
        out_type=jax.ShapeDtypeStruct((NC, NPAD, K), _F32),
        mesh=_mesh(),
        scratch_types=[
            pltpu.VMEM_SHARED((NPAD, K), _F32),
            pltpu.VMEM((EPW,), jnp.int32),
            pltpu.VMEM((EPW,), jnp.int32),
            pltpu.VMEM((NBUF, CH, K), _F32),
            pltpu.VMEM((RPS, K), _F32),
            pltpu.SemaphoreType.DMA,
            pltpu.SemaphoreType.DMA,
        ],
        compiler_params=pltpu.CompilerParams(
            use_tc_tiling_on_sc=False, needs_layout_passes=False),
    )


def _tc1_body(x_ref, w_ref, y_ref):
    y_ref[pl.ds(0, N), :] = jnp.dot(x_ref[...], w_ref[...],
                                    preferred_element_type=_F32)


def _tc2_body(p, xw, dinv, b1, w2, y2_ref):
    d = dinv[...]
    h = jnp.maximum(d * (p[0] + p[1] + d * xw[...]) + b1[...], 0.0)
    y2_ref[...] = jnp.dot(h, w2[...], preferred_element_type=_F32) * d


def _tc3_body(q, y2, dinv, b2, out_ref):
    z = dinv[...] * (q[0] + q[1] + y2[...]) + b2[...]
    m = jnp.max(z, axis=1, keepdims=True)
    e = jnp.exp(z - m)
    lse = jnp.log(jnp.sum(e, axis=1, keepdims=True)) + m
    out_ref[...] = (z - lse)[:N, :OUT]


def kernel(x, edge_index, W1, b1, W2, b2):
    ei_f = edge_index.reshape(2 * E)

    xw = pl.pallas_call(
        _tc1_body,
        out_shape=jax.ShapeDtypeStruct((NPAD, H), _F32),
    )(x, W1)

    p, dinv = _get_layer1()(xw, ei_f)
    dinv2 = dinv.reshape(NPAD, 1)

    w2p = jnp.pad(W2, ((0, 0), (0, 48 - OUT)))
    b2p = jnp.concatenate([b2, jnp.full((48 - OUT,), -1e30, _F32)])
    y2 = pl.pallas_call(
        _tc2_body,
        out_shape=jax.ShapeDtypeStruct((NPAD, 48), _F32),
    )(p, xw, dinv2, b1.reshape(1, H), w2p)

    q = _make_scatter(48)(y2, ei_f)

    return pl.pallas_call(
        _tc3_body,
        out_shape=jax.ShapeDtypeStruct((N, OUT), _F32),
    )(q, y2, dinv2, b2p.reshape(1, 48))
